# baseline (speedup 1.0000x reference)
import functools

import jax
import jax.numpy as jnp
from jax import lax
from jax.experimental import pallas as pl
from jax.experimental.pallas import tpu as pltpu
from jax.experimental.pallas import tpu_sc as plsc

_B = 128
_S = 512
_WE = 128
_OE = 512
_NC = 2
_NS = 16
_NW = _NC * _NS
_TOT = _B * _S
_CHUNK = 256

_NSTAGE = 4
_BPST = _B // _NSTAGE
_BB = 4


def _sc_gather(word_table, ids_flat, tot):
    rpw = tot // _NW
    nch = rpw // _CHUNK
    mesh = plsc.VectorSubcoreMesh(core_axis_name="c", subcore_axis_name="s")

    @functools.partial(
        pl.kernel,
        out_type=jax.ShapeDtypeStruct((tot, _WE), jnp.float32),
        mesh=mesh,
        scratch_types=[
            pltpu.VMEM((rpw,), jnp.int32),
            pltpu.VMEM((_CHUNK, _WE), jnp.float32),
            pltpu.VMEM((_CHUNK, _WE), jnp.float32),
            pltpu.SemaphoreType.DMA,
            pltpu.SemaphoreType.DMA,
        ],
    )
    def k(table_hbm, idx_hbm, out_hbm, idx_v, buf0, buf1, sem0, sem1):
        wid = lax.axis_index("s") * _NC + lax.axis_index("c")
        base = wid * rpw
        pltpu.sync_copy(idx_hbm.at[pl.ds(base, rpw)], idx_v)
        bufs = (buf0, buf1)
        sems = (sem0, sem1)
        copies = [None] * nch
        for ch in range(nch):
            copies[ch] = pltpu.async_copy(
                table_hbm.at[idx_v.at[pl.ds(ch * _CHUNK, _CHUNK)]],
                bufs[ch % 2], sems[ch % 2])
            if ch >= 1:
                copies[ch - 1].wait()
                pltpu.sync_copy(
                    bufs[(ch - 1) % 2],
                    out_hbm.at[pl.ds(base + (ch - 1) * _CHUNK, _CHUNK)])
        copies[nch - 1].wait()
        pltpu.sync_copy(
            bufs[(nch - 1) % 2],
            out_hbm.at[pl.ds(base + (nch - 1) * _CHUNK, _CHUNK)])

    return k(word_table, ids_flat)


def _tc_body(we_ref, tt_ref, k_ref, pos_ref, prm_ref, *rest):
    out_ref = rest[-1]
    we = we_ref[...].reshape(_BB * _S, _WE)
    left = pltpu.roll(we, _BB * _S - 1, 0)
    right = pltpu.roll(we, 1, 0)
    ridx = lax.broadcasted_iota(jnp.int32, (_BB * _S, 1), 0)
    smod = lax.rem(ridx, _S)
    left = jnp.where(smod != _S - 1, left, 0.0)
    right = jnp.where(smod != 0, right, 0.0)
    kk = k_ref[...]
    p = jnp.dot(left, kk[:_WE], preferred_element_type=jnp.float32)
    p = p + jnp.dot(we, kk[_WE:2 * _WE], preferred_element_type=jnp.float32)
    p = p + jnp.dot(right, kk[2 * _WE:], preferred_element_type=jnp.float32)
    prm = prm_ref[...]
    bias = prm[0:1]
    gamma = prm[1:2]
    beta = prm[2:3]
    t0 = prm[3:4]
    dd = prm[4:5] - prm[3:4]
    p = p + bias + t0
    p3 = p.reshape(_BB, _S, _OE)
    pos = pos_ref[...]
    ttT = jnp.transpose(tt_ref[0].astype(jnp.float32))
    for b in range(_BB):
        col = ttT[:, b:b + 1]
        out_ref[b] = (p3[b] + pos + col * dd) * gamma + beta


def _tc_stage(we3, ttf3, proj_kernel, pos_table, prm, prev, stage):
    blk0 = stage * (_BPST // _BB)
    in_specs = [
        pl.BlockSpec((_BB, _S, _WE), lambda i: (i, 0, 0)),
        pl.BlockSpec((1, _BB, _S), lambda i: (i, 0, 0)),
        pl.BlockSpec((3 * _WE, _OE), lambda i: (0, 0)),
        pl.BlockSpec((_S, _OE), lambda i: (0, 0)),
        pl.BlockSpec((8, _OE), lambda i: (0, 0)),
    ]
    args = [we3, ttf3, proj_kernel, pos_table, prm]
    aliases = {}
    if prev is not None:
        in_specs.append(pl.BlockSpec(memory_space=pl.ANY))
        args.append(prev)
        aliases = {5: 0}
    return pl.pallas_call(
        _tc_body,
        grid=(_BPST // _BB,),
        in_specs=in_specs,
        out_specs=pl.BlockSpec((_BB, _S, _OE), lambda i: (i + blk0, 0, 0)),
        out_shape=jax.ShapeDtypeStruct((_B, _S, _OE), jnp.float32),
        input_output_aliases=aliases,
    )(*args)


def kernel(input_ids, token_type_ids, word_table, proj_kernel, proj_bias,
           pos_table, type_table, norm_gamma, norm_beta):
    ids_flat = input_ids.reshape(_TOT)
    prm = jnp.concatenate(
        [proj_bias[None], norm_gamma[None], norm_beta[None],
         type_table, jnp.zeros((3, _OE), jnp.float32)], axis=0)
    pos = pos_table[:_S]
    tps = _BPST * _S
    wes = [
        _sc_gather(word_table, ids_flat[st * tps:(st + 1) * tps], tps)
        for st in range(_NSTAGE)
    ]
    out = None
    for st in range(_NSTAGE):
        we3 = wes[st].reshape(_BPST, _S, _WE)
        tt3 = token_type_ids[st * _BPST:(st + 1) * _BPST].reshape(
            _BPST // _BB, _BB, _S)
        out = _tc_stage(we3, tt3, proj_kernel, pos, prm, out, st)
    return out


# speedup vs baseline: 1.0246x; 1.0246x over previous
import functools

import jax
import jax.numpy as jnp
from jax import lax
from jax.experimental import pallas as pl
from jax.experimental.pallas import tpu as pltpu
from jax.experimental.pallas import tpu_sc as plsc

_B = 128
_S = 512
_WE = 128
_OE = 512
_NC = 2
_NS = 16
_NW = _NC * _NS
_TOT = _B * _S
_CHUNK = 256

_STAGES = (32, 48, 48)
_BB = 4


def _sc_gather(word_table, ids_flat, tot):
    rpw = tot // _NW
    nch = rpw // _CHUNK
    mesh = plsc.VectorSubcoreMesh(core_axis_name="c", subcore_axis_name="s")

    @functools.partial(
        pl.kernel,
        out_type=jax.ShapeDtypeStruct((tot, _WE), jnp.float32),
        mesh=mesh,
        scratch_types=[
            pltpu.VMEM((rpw,), jnp.int32),
            pltpu.VMEM((_CHUNK, _WE), jnp.float32),
            pltpu.VMEM((_CHUNK, _WE), jnp.float32),
            pltpu.SemaphoreType.DMA,
            pltpu.SemaphoreType.DMA,
        ],
    )
    def k(table_hbm, idx_hbm, out_hbm, idx_v, buf0, buf1, sem0, sem1):
        wid = lax.axis_index("s") * _NC + lax.axis_index("c")
        base = wid * rpw
        pltpu.sync_copy(idx_hbm.at[pl.ds(base, rpw)], idx_v)
        bufs = (buf0, buf1)
        sems = (sem0, sem1)
        copies = [None] * nch
        for ch in range(nch):
            copies[ch] = pltpu.async_copy(
                table_hbm.at[idx_v.at[pl.ds(ch * _CHUNK, _CHUNK)]],
                bufs[ch % 2], sems[ch % 2])
            if ch >= 1:
                copies[ch - 1].wait()
                pltpu.sync_copy(
                    bufs[(ch - 1) % 2],
                    out_hbm.at[pl.ds(base + (ch - 1) * _CHUNK, _CHUNK)])
        copies[nch - 1].wait()
        pltpu.sync_copy(
            bufs[(nch - 1) % 2],
            out_hbm.at[pl.ds(base + (nch - 1) * _CHUNK, _CHUNK)])

    return k(word_table, ids_flat)


def _tc_body(we_ref, tt_ref, k_ref, pos_ref, prm_ref, *rest):
    out_ref = rest[-1]
    we = we_ref[...].reshape(_BB * _S, _WE)
    left = pltpu.roll(we, _BB * _S - 1, 0)
    right = pltpu.roll(we, 1, 0)
    ridx = lax.broadcasted_iota(jnp.int32, (_BB * _S, 1), 0)
    smod = lax.rem(ridx, _S)
    left = jnp.where(smod != _S - 1, left, 0.0)
    right = jnp.where(smod != 0, right, 0.0)
    kk = k_ref[...]
    p = jnp.dot(left, kk[:_WE], preferred_element_type=jnp.float32)
    p = p + jnp.dot(we, kk[_WE:2 * _WE], preferred_element_type=jnp.float32)
    p = p + jnp.dot(right, kk[2 * _WE:], preferred_element_type=jnp.float32)
    prm = prm_ref[...]
    bias = prm[0:1]
    gamma = prm[1:2]
    beta = prm[2:3]
    t0 = prm[3:4]
    dd = prm[4:5] - prm[3:4]
    p = p + bias + t0
    p3 = p.reshape(_BB, _S, _OE)
    pos = pos_ref[...]
    ttT = jnp.transpose(tt_ref[0].astype(jnp.float32))
    for b in range(_BB):
        col = ttT[:, b:b + 1]
        out_ref[b] = (p3[b] + pos + col * dd) * gamma + beta


def _tc_stage(we3, ttf3, proj_kernel, pos_table, prm, prev, b0, nb):
    blk0 = b0 // _BB
    in_specs = [
        pl.BlockSpec((_BB, _S, _WE), lambda i: (i, 0, 0)),
        pl.BlockSpec((1, _BB, _S), lambda i: (i, 0, 0)),
        pl.BlockSpec((3 * _WE, _OE), lambda i: (0, 0)),
        pl.BlockSpec((_S, _OE), lambda i: (0, 0)),
        pl.BlockSpec((8, _OE), lambda i: (0, 0)),
    ]
    args = [we3, ttf3, proj_kernel, pos_table, prm]
    aliases = {}
    if prev is not None:
        in_specs.append(pl.BlockSpec(memory_space=pl.ANY))
        args.append(prev)
        aliases = {5: 0}
    return pl.pallas_call(
        _tc_body,
        grid=(nb // _BB,),
        in_specs=in_specs,
        out_specs=pl.BlockSpec((_BB, _S, _OE), lambda i: (i + blk0, 0, 0)),
        out_shape=jax.ShapeDtypeStruct((_B, _S, _OE), jnp.float32),
        input_output_aliases=aliases,
    )(*args)


def kernel(input_ids, token_type_ids, word_table, proj_kernel, proj_bias,
           pos_table, type_table, norm_gamma, norm_beta):
    ids_flat = input_ids.reshape(_TOT)
    prm = jnp.concatenate(
        [proj_bias[None], norm_gamma[None], norm_beta[None],
         type_table, jnp.zeros((3, _OE), jnp.float32)], axis=0)
    pos = pos_table[:_S]
    offs = [0]
    for nb in _STAGES:
        offs.append(offs[-1] + nb)
    wes = [
        _sc_gather(word_table, ids_flat[b0 * _S:(b0 + nb) * _S], nb * _S)
        for b0, nb in zip(offs, _STAGES)
    ]
    out = None
    for st, (b0, nb) in enumerate(zip(offs, _STAGES)):
        we3 = wes[st].reshape(nb, _S, _WE)
        tt3 = token_type_ids[b0:b0 + nb].reshape(nb // _BB, _BB, _S)
        out = _tc_stage(we3, tt3, proj_kernel, pos, prm, out, b0, nb)
    return out
